# baseline (speedup 1.0000x reference)
import functools

import jax
import jax.numpy as jnp
from jax import lax
from jax.experimental import pallas as pl
from jax.experimental.pallas import tpu as pltpu
from jax.experimental.pallas import tpu_sc as plsc

VOCAB_N = 1000000
EMBED_N = 64
TOTAL_N = 4096 * 200
NUM_CORES = 2
NUM_SUBCORES = 16
NUM_WORKERS = NUM_CORES * NUM_SUBCORES
PER_WORKER = TOTAL_N // NUM_WORKERS
CHUNK = 128
CHUNKS_PER_WORKER = PER_WORKER // CHUNK
SCALE = 8.0

_MESH = plsc.VectorSubcoreMesh(
    core_axis_name="c", subcore_axis_name="s", num_cores=NUM_CORES,
    num_subcores=NUM_SUBCORES)


@functools.partial(
    pl.kernel,
    out_type=jax.ShapeDtypeStruct((TOTAL_N, EMBED_N), jnp.float32),
    mesh=_MESH,
    scratch_types=[
        pltpu.VMEM((CHUNKS_PER_WORKER, CHUNK), jnp.int32),
        pltpu.VMEM((CHUNK, EMBED_N), jnp.float32),
        pltpu.SemaphoreType.DMA,
    ],
    compiler_params=pltpu.CompilerParams(use_tc_tiling_on_sc=False),
)
def _embed_lookup(idx_hbm, table_hbm, out_hbm, idx_v, rows_v, sem):
    wid = lax.axis_index("s") * NUM_CORES + lax.axis_index("c")
    base = wid * PER_WORKER
    pltpu.sync_copy(idx_hbm.at[wid], idx_v)

    def chunk_body(c, carry):
        pltpu.async_copy(table_hbm.at[idx_v.at[c]], rows_v, sem).wait()

        def row_body(i, rcarry):
            for j in range(EMBED_N // 16):
                sl = pl.ds(j * 16, 16)
                rows_v[i, sl] = rows_v[i, sl] * SCALE
            return rcarry

        lax.fori_loop(0, CHUNK, row_body, 0, unroll=2)
        pltpu.sync_copy(rows_v, out_hbm.at[pl.ds(base + c * CHUNK, CHUNK)])
        return carry

    lax.fori_loop(0, CHUNKS_PER_WORKER, chunk_body, 0)


def kernel(x, table):
    xf = x.reshape(NUM_WORKERS, CHUNKS_PER_WORKER, CHUNK)
    out = _embed_lookup(xf, table)
    return out.reshape(x.shape[0], x.shape[1], EMBED_N)


# speedup vs baseline: 1.1620x; 1.1620x over previous
import functools

import jax
import jax.numpy as jnp
from jax import lax
from jax.experimental import pallas as pl
from jax.experimental.pallas import tpu as pltpu
from jax.experimental.pallas import tpu_sc as plsc

VOCAB_N = 1000000
EMBED_N = 64
TOTAL_N = 4096 * 200
NUM_CORES = 2
NUM_SUBCORES = 16
NUM_WORKERS = NUM_CORES * NUM_SUBCORES
PER_WORKER = TOTAL_N // NUM_WORKERS
CHUNK = 128
SUP = 512
GPB = SUP // CHUNK
NSUP = PER_WORKER // SUP
NBUF = 2
SCALE = 8.0

_MESH = plsc.VectorSubcoreMesh(
    core_axis_name="c", subcore_axis_name="s", num_cores=NUM_CORES,
    num_subcores=NUM_SUBCORES)


@functools.partial(
    pl.kernel,
    out_type=jax.ShapeDtypeStruct((TOTAL_N, EMBED_N), jnp.float32),
    mesh=_MESH,
    scratch_types=[
        pltpu.VMEM((PER_WORKER // CHUNK, CHUNK), jnp.int32),
        pltpu.VMEM((SUP, EMBED_N), jnp.float32),
        pltpu.VMEM((SUP, EMBED_N), jnp.float32),
        pltpu.SemaphoreType.DMA,
        pltpu.SemaphoreType.DMA,
        pltpu.SemaphoreType.DMA,
        pltpu.SemaphoreType.DMA,
    ],
    compiler_params=pltpu.CompilerParams(use_tc_tiling_on_sc=False),
)
def _embed_lookup(idx_hbm, table_hbm, out_hbm, idx_v, rows0, rows1,
                  gsem0, gsem1, osem0, osem1):
    wid = lax.axis_index("s") * NUM_CORES + lax.axis_index("c")
    base = wid * PER_WORKER
    rows = (rows0, rows1)
    gsem = (gsem0, gsem1)
    osem = (osem0, osem1)

    pltpu.sync_copy(idx_hbm.at[wid], idx_v)

    def gather_part(s, b, k):
        return (table_hbm.at[idx_v.at[s * GPB + k]],
                rows[b].at[pl.ds(k * CHUNK, CHUNK)], gsem[b])

    def out_part(s, b):
        return rows[b], out_hbm.at[pl.ds(base + s * SUP, SUP)], osem[b]

    def scale_slice(b, k):
        def row_body(i, carry):
            for j in range(EMBED_N // 16):
                sl = pl.ds(j * 16, 16)
                rows[b][i, sl] = rows[b][i, sl] * SCALE
            return carry
        lax.fori_loop(k * CHUNK, (k + 1) * CHUNK, row_body, 0, unroll=2)

    for k in range(GPB):
        pltpu.async_copy(*gather_part(0, 0, k))

    @pl.loop(0, NSUP, step=NBUF)
    def _(s0):
        for b in range(NBUF):
            s = s0 + b
            bn = (b + 1) % NBUF

            @pl.when(s + 1 < NSUP)
            def _():
                @pl.when(s + 1 >= NBUF)
                def _():
                    pltpu.make_async_copy(*out_part(s + 1 - NBUF, bn)).wait()
                for k in range(GPB):
                    pltpu.async_copy(*gather_part(s + 1, bn, k))

            for k in range(GPB):
                pltpu.make_async_copy(*gather_part(s, b, k)).wait()
                scale_slice(b, k)

            pltpu.async_copy(*out_part(s, b))

    for b in range(NBUF):
        pltpu.make_async_copy(*out_part(NSUP - NBUF + b, b)).wait()


def kernel(x, table):
    xf = x.reshape(NUM_WORKERS, PER_WORKER // CHUNK, CHUNK)
    out = _embed_lookup(xf, table)
    return out.reshape(x.shape[0], x.shape[1], EMBED_N)
